# baseline (speedup 1.0000x reference)
import math

import jax
import jax.numpy as jnp
from jax import lax
from jax.experimental import pallas as pl
from jax.experimental.pallas import tpu as pltpu
from jax.experimental.pallas import tpu_sc as plsc

_VOCAB = 1000000
_HIDDEN = 64
_B = 1024
_L = 200
_SCALE = math.sqrt(_HIDDEN)

_NC = 2
_NS = 16
_NW = _NC * _NS
_SEQ_PER_W = _B // _NW
_RING = 8


def _body(source_hbm, token_hbm, pos_hbm, out_hbm,
          pos_v, idx_v, blk, obuf, sems):
  wid = lax.axis_index("s") * _NC + lax.axis_index("c")

  pltpu.sync_copy(pos_hbm.at[pl.ds(0, _L)], pos_v)

  def scalar_idx(row):
    return idx_v[pl.ds(row, 16)][0]

  def issue(row, slot):
    r = scalar_idx(row)
    base = pl.multiple_of((r >> 3) * 8, 8)
    pltpu.async_copy(token_hbm.at[pl.ds(base, 8)], blk[slot], sems[slot])

  def wait_slot(slot):
    pltpu.make_async_copy(token_hbm.at[pl.ds(0, 8)], blk[slot],
                          sems[slot]).wait()

  def seq_body(s, carry):
    seq = wid * _SEQ_PER_W + s
    base = pl.multiple_of(seq * _L, 8)
    pltpu.sync_copy(source_hbm.at[pl.ds(base, _L)], idx_v.at[pl.ds(0, _L)])

    for j in range(_RING):
      issue(j, j)

    def row_block(i, c2):
      for j in range(_RING):
        row = i * _RING + j
        wait_slot(j)
        sub = scalar_idx(row) & 7
        for c in range(_HIDDEN // 16):
          sl = pl.ds(c * 16, 16)
          obuf[row, sl] = blk[j][sub, sl] * _SCALE + pos_v[row, sl]

        @pl.when(row + _RING < _L)
        def _():
          issue(row + _RING, j)

      return c2

    lax.fori_loop(0, _L // _RING, row_block, 0)
    pltpu.sync_copy(obuf, out_hbm.at[seq])
    return carry

  lax.fori_loop(0, _SEQ_PER_W, seq_body, 0)


@jax.jit
def kernel(source, token_table, pos_table):
  mesh = plsc.VectorSubcoreMesh(core_axis_name="c", subcore_axis_name="s",
                                num_cores=_NC, num_subcores=_NS)
  run = pl.kernel(
      _body,
      out_type=jax.ShapeDtypeStruct((_B, _L, _HIDDEN), jnp.float32),
      mesh=mesh,
      scratch_types=[
          pltpu.VMEM((_L, _HIDDEN), jnp.float32),
          pltpu.VMEM((_L + 16,), jnp.int32),
          [pltpu.VMEM((8, _HIDDEN), jnp.float32)] * _RING,
          pltpu.VMEM((_L, _HIDDEN), jnp.float32),
          [pltpu.SemaphoreType.DMA] * _RING,
      ],
  )
  return run(source.reshape(-1), token_table, pos_table)
